# Initial kernel scaffold; baseline (speedup 1.0000x reference)
import jax
import jax.numpy as jnp
from jax.experimental import pallas as pl


def kernel(x, pos, batch, W1, b1, W2, b2, W3, b3):
    raise NotImplementedError("write your pallas kernel here")


# speedup vs baseline: 4.4032x; 4.4032x over previous
import jax
import jax.numpy as jnp
import numpy as np
from jax import lax
from jax.experimental import pallas as pl
from jax.experimental.pallas import tpu as pltpu
from jax.experimental.pallas import tpu_sc as plsc

B = 64
S = 4096
KP = 1024
K = 32
NC, NS, L = 2, 16, 16
NW = NC * NS
BPW = B // NW
FINF = np.float32(3.0e38)
MAXBITS = np.int32(0x7F7FFFFF)


def _cutoff_body(px_ref, py_ref, db_ref, t_ref, c_ref):
    px = px_ref[...]
    py = py_ref[...]
    dist = jnp.sqrt(px * px + py * py)
    db = lax.bitcast_convert_type(dist, jnp.int32)
    db_ref[...] = db

    def vstep(_, st):
        lo, hi = st
        mid = lo + (hi - lo) // 2
        cnt = jnp.sum((db <= mid).astype(jnp.int32), axis=1, keepdims=True)
        ge = cnt >= KP
        return jnp.where(ge, lo, mid), jnp.where(ge, mid, hi)

    lo0 = jnp.full((B, 1), -1, jnp.int32)
    hi0 = jnp.full((B, 1), MAXBITS, jnp.int32)
    _, t = lax.fori_loop(0, 31, vstep, (lo0, hi0))

    iota = lax.broadcasted_iota(jnp.int32, (B, S), 1)
    lt_cnt = jnp.sum((db < t).astype(jnp.int32), axis=1, keepdims=True)
    eq = db == t

    def istep(_, st):
        lo, hi = st
        mid = lo + (hi - lo) // 2
        cnt = lt_cnt + jnp.sum((eq & (iota <= mid)).astype(jnp.int32), axis=1,
                               keepdims=True)
        ge = cnt >= KP
        return jnp.where(ge, lo, mid), jnp.where(ge, mid, hi)

    clo0 = jnp.full((B, 1), -1, jnp.int32)
    chi0 = jnp.full((B, 1), S - 1, jnp.int32)
    _, c = lax.fori_loop(0, 12, istep, (clo0, chi0))

    t_ref[...] = jnp.broadcast_to(t, (B, 128))
    c_ref[...] = jnp.broadcast_to(c, (B, 128))


_cutoff = pl.pallas_call(
    _cutoff_body,
    out_shape=[
        jax.ShapeDtypeStruct((B, S), jnp.int32),
        jax.ShapeDtypeStruct((B, 128), jnp.int32),
        jax.ShapeDtypeStruct((B, 128), jnp.int32),
    ],
)


def _fps_body(px, py, pz, dbm, taux, caux, x_hbm,
              idx_out, xsel_out,
              xs, ys, zs, dbv, cx, cy, cz, cdb, cgi,
              mind, gsel, trow, crow, sem):
    wid = lax.axis_index("s") * NC + lax.axis_index("c")
    lane = lax.broadcasted_iota(jnp.int32, (L,), 0)

    for bi in range(BPW):
        b = wid * BPW + bi
        pltpu.sync_copy(px.at[b], xs)
        pltpu.sync_copy(py.at[b], ys)
        pltpu.sync_copy(pz.at[b], zs)
        pltpu.sync_copy(dbm.at[b], dbv)
        pltpu.sync_copy(taux.at[b], trow)
        pltpu.sync_copy(caux.at[b], crow)
        tv = trow[pl.ds(0, L)]
        cv = crow[pl.ds(0, L)]

        def comp_step(s, off):
            d16 = dbv[pl.ds(s * L, L)]
            gi = lane + jnp.full((L,), s * L, jnp.int32)
            m = (d16 < tv) | ((d16 == tv) & (gi <= cv))
            mi = m.astype(jnp.int32)
            inc = plsc.cumsum(mi)
            idx = (inc - mi) + jnp.full((L,), off, jnp.int32)
            plsc.store_scatter(cx, [idx], xs[pl.ds(s * L, L)], mask=m)
            plsc.store_scatter(cy, [idx], ys[pl.ds(s * L, L)], mask=m)
            plsc.store_scatter(cz, [idx], zs[pl.ds(s * L, L)], mask=m)
            plsc.store_scatter(cdb, [idx], d16, mask=m)
            plsc.store_scatter(cgi, [idx], gi, mask=m)
            return off + inc[L - 1]

        lax.fori_loop(0, S // L, comp_step, np.int32(0), unroll=4)

        def amin_step(s, st):
            bmin, bidx = st
            d16 = cdb[pl.ds(s * L, L)]
            pos16 = lane + jnp.full((L,), s * L, jnp.int32)
            better = d16 < bmin
            return jnp.where(better, d16, bmin), jnp.where(better, pos16, bidx)

        bmin, bidx = lax.fori_loop(
            0, KP // L, amin_step,
            (jnp.full((L,), MAXBITS, jnp.int32), jnp.zeros((L,), jnp.int32)),
            unroll=4)
        kpv = jnp.full((L,), KP, jnp.int32)
        mnv = jnp.full((L,), jnp.min(bmin), jnp.int32)
        c0 = jnp.min(jnp.where(bmin == mnv, bidx, kpv))

        def minit_step(s, carry):
            mind[pl.ds(s * L, L)] = jnp.full((L,), FINF, jnp.float32)
            return carry

        lax.fori_loop(0, KP // L, minit_step, np.int32(0), unroll=4)
        g0 = jnp.full((L,), cgi[pl.ds(c0, L)][0], jnp.int32)
        lane0 = lane == jnp.zeros((L,), jnp.int32)
        plsc.store_scatter(gsel, [jnp.zeros((L,), jnp.int32)], g0, mask=lane0)

        def fps_step(k, sp):
            lx = jnp.full((L,), cx[pl.ds(sp, L)][0], jnp.float32)
            ly = jnp.full((L,), cy[pl.ds(sp, L)][0], jnp.float32)
            lz = jnp.full((L,), cz[pl.ds(sp, L)][0], jnp.float32)

            def scan_step(s, st):
                rmax, ridx = st
                dx = cx[pl.ds(s * L, L)] - lx
                dy = cy[pl.ds(s * L, L)] - ly
                dz = cz[pl.ds(s * L, L)] - lz
                dd = dx * dx + dy * dy + dz * dz
                nm = jnp.minimum(mind[pl.ds(s * L, L)], dd)
                mind[pl.ds(s * L, L)] = nm
                better = nm > rmax
                pos16 = lane + jnp.full((L,), s * L, jnp.int32)
                return (jnp.where(better, nm, rmax),
                        jnp.where(better, pos16, ridx))

            rmax, ridx = lax.fori_loop(
                0, KP // L, scan_step,
                (jnp.full((L,), -FINF, jnp.float32), jnp.zeros((L,), jnp.int32)),
                unroll=8)
            mxv = jnp.full((L,), jnp.max(rmax), jnp.float32)
            kpv2 = jnp.full((L,), KP, jnp.int32)
            nxt = jnp.min(jnp.where(rmax == mxv, ridx, kpv2))
            gval = jnp.full((L,), cgi[pl.ds(nxt, L)][0], jnp.int32)
            plsc.store_scatter(gsel, [jnp.full((L,), k, jnp.int32)], gval,
                               mask=lane0)
            return nxt

        lax.fori_loop(1, K, fps_step, c0)

        pltpu.sync_copy(gsel, idx_out.at[b])
        descs = []
        for k in range(K):
            r = gsel[pl.ds(k, L)][0] + b * S
            descs.append(
                pltpu.async_copy(x_hbm.at[r], xsel_out.at[b * K + k], sem))
        for d in descs:
            d.wait()


_fps = pl.kernel(
    _fps_body,
    out_type=[
        jax.ShapeDtypeStruct((B, 128), jnp.int32),
        jax.ShapeDtypeStruct((B * K, 32), jnp.float32),
    ],
    mesh=plsc.VectorSubcoreMesh(core_axis_name="c", subcore_axis_name="s",
                                num_cores=NC, num_subcores=NS),
    scratch_types=[
        pltpu.VMEM((S,), jnp.float32),
        pltpu.VMEM((S,), jnp.float32),
        pltpu.VMEM((S,), jnp.float32),
        pltpu.VMEM((S,), jnp.int32),
        pltpu.VMEM((KP + L,), jnp.float32),
        pltpu.VMEM((KP + L,), jnp.float32),
        pltpu.VMEM((KP + L,), jnp.float32),
        pltpu.VMEM((KP + L,), jnp.int32),
        pltpu.VMEM((KP + L,), jnp.int32),
        pltpu.VMEM((KP,), jnp.float32),
        pltpu.VMEM((128,), jnp.int32),
        pltpu.VMEM((128,), jnp.int32),
        pltpu.VMEM((128,), jnp.int32),
        pltpu.SemaphoreType.DMA,
    ],
    compiler_params=pltpu.CompilerParams(needs_layout_passes=False),
)


def _mlp_body(x_ref, w1_ref, b1_ref, w2_ref, b2_ref, w3_ref, b3_ref, out_ref):
    xv = x_ref[...]
    h = jnp.dot(xv, w1_ref[...], preferred_element_type=jnp.float32) + b1_ref[...]
    h = jnp.maximum(h, 0.0)
    h = jnp.dot(h, w2_ref[...], preferred_element_type=jnp.float32) + b2_ref[...]
    h = jnp.maximum(h, 0.0)
    s = jnp.dot(h, w3_ref[...], preferred_element_type=jnp.float32) + b3_ref[...]
    out_ref[...] = jnp.maximum(s, 0.0) + jnp.log1p(jnp.exp(-jnp.abs(s)))


_mlp = pl.pallas_call(
    _mlp_body,
    out_shape=jax.ShapeDtypeStruct((B * K, 1), jnp.float32),
)


def kernel(x, pos, batch, W1, b1, W2, b2, W3, b3):
    pos3 = pos.reshape(B, S, 3)
    px = pos3[:, :, 0]
    py = pos3[:, :, 1]
    pz = pos3[:, :, 2]
    dbits, taux, caux = _cutoff(px, py)
    idx_pad, xsel = _fps(px, py, pz, dbits, taux, caux, x)
    s = _mlp(xsel, W1, b1.reshape(1, 16),
             W2, b2.reshape(1, 8), W3, b3.reshape(1, 1))
    return s.reshape(B, K), idx_pad[:, :K]
